# initial kernel scaffold (unmeasured)
import functools

import jax
import jax.numpy as jnp
from jax import lax
from jax.experimental import pallas as pl
from jax.experimental.pallas import tpu as pltpu

B, S, H, Dh, Dr = 4, 256, 32, 128, 64
D = 4096
DC = 256
DC_SH = 128
M = B * S

_CompilerParams = getattr(pltpu, "CompilerParams", None) or getattr(
    pltpu, "TPUCompilerParams"
)


def _exchange_body(
    x_ref, wdkv_ref, wuk_ref, wuv_ref,
    xbf_ref, c_ref, wuk2_ref, wuv2_ref,
    c_loc, wukbf, wuvbf,
    c_rem, wuk_rem, wuv_rem,
    send_sems, recv_sems,
):
    my_x = lax.axis_index("x")
    my_y = lax.axis_index("y")
    my_z = lax.axis_index("z")
    partner = (1 - my_x, my_y, my_z)

    barrier = pltpu.get_barrier_semaphore()
    pl.semaphore_signal(
        barrier, inc=1, device_id=partner, device_id_type=pl.DeviceIdType.MESH
    )
    pl.semaphore_wait(barrier, 1)

    xbf_ref[...] = x_ref[...].astype(jnp.bfloat16)
    c_loc[...] = jnp.dot(
        xbf_ref[...], wdkv_ref[...].astype(jnp.bfloat16),
        preferred_element_type=jnp.float32,
    ).astype(jnp.bfloat16)
    wukbf[...] = wuk_ref[...].astype(jnp.bfloat16)
    wuvbf[...] = wuv_ref[...].astype(jnp.bfloat16)

    rdmas = []
    for i, (src, dst) in enumerate(
        [(c_loc, c_rem), (wukbf, wuk_rem), (wuvbf, wuv_rem)]
    ):
        rdma = pltpu.make_async_remote_copy(
            src_ref=src,
            dst_ref=dst,
            send_sem=send_sems.at[i],
            recv_sem=recv_sems.at[i],
            device_id=partner,
            device_id_type=pl.DeviceIdType.MESH,
        )
        rdma.start()
        rdmas.append(rdma)

    c_ref[:, 0:DC_SH] = c_loc[...]
    wuk2_ref[0:DC_SH, :] = wukbf[...]
    wuv2_ref[0:DC_SH, :] = wuvbf[...]

    for rdma in rdmas:
        rdma.wait()

    c_ref[:, DC_SH:DC] = c_rem[...]
    wuk2_ref[DC_SH:DC, :] = wuk_rem[...]
    wuv2_ref[DC_SH:DC, :] = wuv_rem[...]


def _exchange(x2, wdkv, wuk, wuv):
    vmem = pl.BlockSpec(memory_space=pltpu.VMEM)
    return pl.pallas_call(
        _exchange_body,
        out_shape=[
            jax.ShapeDtypeStruct((M, D), jnp.bfloat16),
            jax.ShapeDtypeStruct((M, DC), jnp.bfloat16),
            jax.ShapeDtypeStruct((DC, D), jnp.bfloat16),
            jax.ShapeDtypeStruct((DC, D), jnp.bfloat16),
        ],
        in_specs=[vmem] * 4,
        out_specs=[vmem] * 4,
        scratch_shapes=[
            pltpu.VMEM((M, DC_SH), jnp.bfloat16),
            pltpu.VMEM((DC_SH, D), jnp.bfloat16),
            pltpu.VMEM((DC_SH, D), jnp.bfloat16),
            pltpu.VMEM((M, DC_SH), jnp.bfloat16),
            pltpu.VMEM((DC_SH, D), jnp.bfloat16),
            pltpu.VMEM((DC_SH, D), jnp.bfloat16),
            pltpu.SemaphoreType.DMA((3,)),
            pltpu.SemaphoreType.DMA((3,)),
        ],
        compiler_params=_CompilerParams(collective_id=0),
    )(x2, wdkv, wuk, wuv)


def _matmul_body(a_ref, w_ref, o_ref, *, out_dtype):
    a = a_ref[...]
    w = w_ref[...]
    if a.dtype != jnp.bfloat16:
        a = a.astype(jnp.bfloat16)
    if w.dtype != jnp.bfloat16:
        w = w.astype(jnp.bfloat16)
    o_ref[...] = jnp.dot(a, w, preferred_element_type=jnp.float32).astype(
        out_dtype
    )


def _matmul(a, w, bn, out_dtype):
    m, k = a.shape
    _, n = w.shape
    assert n % bn == 0
    return pl.pallas_call(
        functools.partial(_matmul_body, out_dtype=out_dtype),
        grid=(n // bn,),
        in_specs=[
            pl.BlockSpec((m, k), lambda j: (0, 0)),
            pl.BlockSpec((k, bn), lambda j: (0, j)),
        ],
        out_specs=pl.BlockSpec((m, bn), lambda j: (0, j)),
        out_shape=jax.ShapeDtypeStruct((m, n), out_dtype),
    )(a, w)


def _attn_body(q_ref, k_ref, qr_ref, kr_ref, v_ref, o_ref):
    q = q_ref[0]
    k = k_ref[0]
    qr = qr_ref[0]
    kr = kr_ref[0]
    v = v_ref[0]
    dims = (((1,), (1,)), ((), ()))
    s = lax.dot_general(q, k, dims, preferred_element_type=jnp.float32)
    s = s + lax.dot_general(qr, kr, dims, preferred_element_type=jnp.float32)
    s = s * (Dh + Dr) ** -0.5
    mx = jnp.max(s, axis=1, keepdims=True)
    p = jnp.exp(s - mx)
    p = p / jnp.sum(p, axis=1, keepdims=True)
    o = jnp.dot(
        p.astype(jnp.bfloat16), v, preferred_element_type=jnp.float32
    )
    o_ref[0] = o.astype(jnp.bfloat16)


def _attention(q3, k3, qr3, kr3, v3):
    return pl.pallas_call(
        _attn_body,
        grid=(B, H),
        in_specs=[
            pl.BlockSpec((1, S, Dh), lambda b, h: (b, 0, h)),
            pl.BlockSpec((1, S, Dh), lambda b, h: (b, 0, h)),
            pl.BlockSpec((1, S, Dr), lambda b, h: (b, 0, h)),
            pl.BlockSpec((1, S, Dr), lambda b, h: (b, 0, 0)),
            pl.BlockSpec((1, S, Dh), lambda b, h: (b, 0, h)),
        ],
        out_specs=pl.BlockSpec((1, S, Dh), lambda b, h: (b, 0, h)),
        out_shape=jax.ShapeDtypeStruct((B, S, H * Dh), jnp.bfloat16),
    )(q3, k3, qr3, kr3, v3)


def kernel(x, Wdkv, Wuk, Wuv, Wq, Wqr, Wkr, Wo):
    x2 = x.reshape(M, D)

    xbf, c2, wuk2, wuv2 = _exchange(x2, Wdkv, Wuk, Wuv)

    k2 = _matmul(c2, wuk2, 512, jnp.bfloat16)
    v2 = _matmul(c2, wuv2, 512, jnp.bfloat16)
    q2 = _matmul(xbf, Wq, 512, jnp.bfloat16)
    qr2 = _matmul(xbf, Wqr, 512, jnp.bfloat16)
    kr2 = _matmul(xbf, Wkr, 64, jnp.bfloat16)

    o3 = _attention(
        q2.reshape(B, S, H * Dh),
        k2.reshape(B, S, H * Dh),
        qr2.reshape(B, S, H * Dr),
        kr2.reshape(B, S, Dr),
        v2.reshape(B, S, H * Dh),
    )

    out2 = _matmul(o3.reshape(M, H * Dh), Wo, 512, jnp.float32)
    return out2.reshape(B, S, D)


# baseline (device time: 282911 ns/iter reference)
import functools

import jax
import jax.numpy as jnp
from jax import lax
from jax.experimental import pallas as pl
from jax.experimental.pallas import tpu as pltpu

B, S, H, Dh, Dr = 4, 256, 32, 128, 64
D = 4096
DC = 256
DC_SH = 128
M = B * S

_CompilerParams = getattr(pltpu, "CompilerParams", None) or getattr(
    pltpu, "TPUCompilerParams"
)


def _exchange_body(
    x_ref, wdkv_ref, wuk_ref, wuv_ref,
    xbf_ref, c_ref, wuk2_ref, wuv2_ref,
    c_loc, wukbf, wuvbf,
    c_rem, wuk_rem, wuv_rem,
    send_sems, recv_sems,
):
    my_x = lax.axis_index("x")
    my_y = lax.axis_index("y")
    my_z = lax.axis_index("z")
    partner = (1 - my_x, my_y, my_z)

    barrier = pltpu.get_barrier_semaphore()
    pl.semaphore_signal(
        barrier, inc=1, device_id=partner, device_id_type=pl.DeviceIdType.MESH
    )
    pl.semaphore_wait(barrier, 1)

    xbf_ref[...] = x_ref[...].astype(jnp.bfloat16)
    c_loc[...] = jnp.dot(
        xbf_ref[...], wdkv_ref[...].astype(jnp.bfloat16),
        preferred_element_type=jnp.float32,
    ).astype(jnp.bfloat16)
    wukbf[...] = wuk_ref[...].astype(jnp.bfloat16)
    wuvbf[...] = wuv_ref[...].astype(jnp.bfloat16)

    rdmas = []
    for i, (src, dst) in enumerate(
        [(c_loc, c_rem), (wukbf, wuk_rem), (wuvbf, wuv_rem)]
    ):
        rdma = pltpu.make_async_remote_copy(
            src_ref=src,
            dst_ref=dst,
            send_sem=send_sems.at[i],
            recv_sem=recv_sems.at[i],
            device_id=partner,
            device_id_type=pl.DeviceIdType.MESH,
        )
        rdma.start()
        rdmas.append(rdma)

    c_ref[:, 0:DC_SH] = c_loc[...]
    wuk2_ref[0:DC_SH, :] = wukbf[...]
    wuv2_ref[0:DC_SH, :] = wuvbf[...]

    for rdma in rdmas:
        rdma.wait()

    c_ref[:, DC_SH:DC] = c_rem[...]
    wuk2_ref[DC_SH:DC, :] = wuk_rem[...]
    wuv2_ref[DC_SH:DC, :] = wuv_rem[...]


def _exchange(x2, wdkv, wuk, wuv):
    vmem = pl.BlockSpec(memory_space=pltpu.VMEM)
    return pl.pallas_call(
        _exchange_body,
        out_shape=[
            jax.ShapeDtypeStruct((M, D), jnp.bfloat16),
            jax.ShapeDtypeStruct((M, DC), jnp.bfloat16),
            jax.ShapeDtypeStruct((DC, D), jnp.bfloat16),
            jax.ShapeDtypeStruct((DC, D), jnp.bfloat16),
        ],
        in_specs=[vmem] * 4,
        out_specs=[vmem] * 4,
        scratch_shapes=[
            pltpu.VMEM((M, DC_SH), jnp.bfloat16),
            pltpu.VMEM((DC_SH, D), jnp.bfloat16),
            pltpu.VMEM((DC_SH, D), jnp.bfloat16),
            pltpu.VMEM((M, DC_SH), jnp.bfloat16),
            pltpu.VMEM((DC_SH, D), jnp.bfloat16),
            pltpu.VMEM((DC_SH, D), jnp.bfloat16),
            pltpu.SemaphoreType.DMA((3,)),
            pltpu.SemaphoreType.DMA((3,)),
        ],
        compiler_params=_CompilerParams(collective_id=0),
    )(x2, wdkv, wuk, wuv)


def _matmul_body(a_ref, w_ref, o_ref, *, out_dtype):
    a = a_ref[...]
    w = w_ref[...]
    if a.dtype != jnp.bfloat16:
        a = a.astype(jnp.bfloat16)
    if w.dtype != jnp.bfloat16:
        w = w.astype(jnp.bfloat16)
    o_ref[...] = jnp.dot(a, w, preferred_element_type=jnp.float32).astype(
        out_dtype
    )


def _matmul(a, w, bn, out_dtype):
    m, k = a.shape
    _, n = w.shape
    assert n % bn == 0
    return pl.pallas_call(
        functools.partial(_matmul_body, out_dtype=out_dtype),
        grid=(n // bn,),
        in_specs=[
            pl.BlockSpec((m, k), lambda j: (0, 0)),
            pl.BlockSpec((k, bn), lambda j: (0, j)),
        ],
        out_specs=pl.BlockSpec((m, bn), lambda j: (0, j)),
        out_shape=jax.ShapeDtypeStruct((m, n), out_dtype),
    )(a, w)


def _attn_body(q_ref, k_ref, qr_ref, kr_ref, v_ref, o_ref):
    q = q_ref[0]
    k = k_ref[0]
    h = pl.program_id(1)
    qrp = qr_ref[0]
    lane = lax.broadcasted_iota(jnp.int32, (S, 2 * Dr), 1)
    qr_m = jnp.where((lane // Dr) == (h % 2), qrp, 0).astype(qrp.dtype)
    kr = kr_ref[0]
    kr_cat = jnp.concatenate([kr, kr], axis=1)
    v = v_ref[0]
    dims = (((1,), (1,)), ((), ()))
    s = lax.dot_general(q, k, dims, preferred_element_type=jnp.float32)
    s = s + lax.dot_general(qr_m, kr_cat, dims, preferred_element_type=jnp.float32)
    s = s * (Dh + Dr) ** -0.5
    mx = jnp.max(s, axis=1, keepdims=True)
    p = jnp.exp(s - mx)
    p = p / jnp.sum(p, axis=1, keepdims=True)
    o = jnp.dot(
        p.astype(jnp.bfloat16), v, preferred_element_type=jnp.float32
    )
    o_ref[0] = o.astype(jnp.bfloat16)


def _attention(q3, k3, qr3, kr3, v3):
    return pl.pallas_call(
        _attn_body,
        grid=(B, H),
        in_specs=[
            pl.BlockSpec((1, S, Dh), lambda b, h: (b, 0, h)),
            pl.BlockSpec((1, S, Dh), lambda b, h: (b, 0, h)),
            pl.BlockSpec((1, S, 2 * Dr), lambda b, h: (b, 0, h // 2)),
            pl.BlockSpec((1, S, Dr), lambda b, h: (b, 0, 0)),
            pl.BlockSpec((1, S, Dh), lambda b, h: (b, 0, h)),
        ],
        out_specs=pl.BlockSpec((1, S, Dh), lambda b, h: (b, 0, h)),
        out_shape=jax.ShapeDtypeStruct((B, S, H * Dh), jnp.bfloat16),
    )(q3, k3, qr3, kr3, v3)


def kernel(x, Wdkv, Wuk, Wuv, Wq, Wqr, Wkr, Wo):
    x2 = x.reshape(M, D)

    xbf, c2, wuk2, wuv2 = _exchange(x2, Wdkv, Wuk, Wuv)

    k2 = _matmul(c2, wuk2, 512, jnp.bfloat16)
    v2 = _matmul(c2, wuv2, 512, jnp.bfloat16)
    q2 = _matmul(xbf, Wq, 512, jnp.bfloat16)
    qr2 = _matmul(xbf, Wqr, 512, jnp.bfloat16)
    kr2 = _matmul(xbf, Wkr, 64, jnp.bfloat16)

    o3 = _attention(
        q2.reshape(B, S, H * Dh),
        k2.reshape(B, S, H * Dh),
        qr2.reshape(B, S, H * Dr),
        kr2.reshape(B, S, Dr),
        v2.reshape(B, S, H * Dh),
    )

    out2 = _matmul(o3.reshape(M, H * Dh), Wo, 512, jnp.float32)
    return out2.reshape(B, S, D)


# device time: 219749 ns/iter; 1.2874x vs baseline; 1.2874x over previous
import functools

import jax
import jax.numpy as jnp
from jax import lax
from jax.experimental import pallas as pl
from jax.experimental.pallas import tpu as pltpu

B, S, H, Dh, Dr = 4, 256, 32, 128, 64
D = 4096
DC = 256
DC_SH = 128
M = B * S

_CompilerParams = getattr(pltpu, "CompilerParams", None) or getattr(
    pltpu, "TPUCompilerParams"
)


_QBLK = 256


def _exq_body(
    x_ref, wdkv_ref, wuk_ref, wuv_ref, wq_ref,
    c_ref, wuk2_ref, wuv2_ref, q_ref,
    c_loc,
    c_rem, wuk_rem, wuv_rem,
    send_sems, recv_sems,
):
    j = pl.program_id(0)
    my_x = lax.axis_index("x")
    my_y = lax.axis_index("y")
    my_z = lax.axis_index("z")
    partner = (1 - my_x, my_y, my_z)

    rdmas = [
        pltpu.make_async_remote_copy(
            src_ref=src,
            dst_ref=dst,
            send_sem=send_sems.at[i],
            recv_sem=recv_sems.at[i],
            device_id=partner,
            device_id_type=pl.DeviceIdType.MESH,
        )
        for i, (src, dst) in enumerate(
            [(wuk_ref, wuk_rem), (wuv_ref, wuv_rem), (c_loc, c_rem)]
        )
    ]

    @pl.when(j == 0)
    def _():
        barrier = pltpu.get_barrier_semaphore()
        pl.semaphore_signal(
            barrier, inc=1, device_id=partner,
            device_id_type=pl.DeviceIdType.MESH,
        )
        pl.semaphore_wait(barrier, 1)

        rdmas[0].start()
        rdmas[1].start()
        c_loc[...] = jnp.dot(
            x_ref[...], wdkv_ref[...], preferred_element_type=jnp.float32
        ).astype(jnp.bfloat16)
        rdmas[2].start()
        c_ref[:, 0:DC_SH] = c_loc[...]
        wuk2_ref[0:DC_SH, :] = wuk_ref[...]
        wuv2_ref[0:DC_SH, :] = wuv_ref[...]

    q_ref[...] = jnp.dot(
        x_ref[...], wq_ref[...].astype(jnp.bfloat16),
        preferred_element_type=jnp.float32,
    ).astype(jnp.bfloat16)

    @pl.when(j == pl.num_programs(0) - 1)
    def _():
        for rdma in rdmas:
            rdma.wait()
        c_ref[:, DC_SH:DC] = c_rem[...]
        wuk2_ref[DC_SH:DC, :] = wuk_rem[...]
        wuv2_ref[DC_SH:DC, :] = wuv_rem[...]


def _exchange_q(x2bf, wdkv_bf, wuk_bf, wuv_bf, wq):
    return pl.pallas_call(
        _exq_body,
        grid=(D // _QBLK,),
        out_shape=[
            jax.ShapeDtypeStruct((M, DC), jnp.bfloat16),
            jax.ShapeDtypeStruct((DC, D), jnp.bfloat16),
            jax.ShapeDtypeStruct((DC, D), jnp.bfloat16),
            jax.ShapeDtypeStruct((M, D), jnp.bfloat16),
        ],
        in_specs=[
            pl.BlockSpec((M, D), lambda j: (0, 0)),
            pl.BlockSpec((D, DC_SH), lambda j: (0, 0)),
            pl.BlockSpec((DC_SH, D), lambda j: (0, 0)),
            pl.BlockSpec((DC_SH, D), lambda j: (0, 0)),
            pl.BlockSpec((D, _QBLK), lambda j: (0, j)),
        ],
        out_specs=[
            pl.BlockSpec((M, DC), lambda j: (0, 0)),
            pl.BlockSpec((DC, D), lambda j: (0, 0)),
            pl.BlockSpec((DC, D), lambda j: (0, 0)),
            pl.BlockSpec((M, _QBLK), lambda j: (0, j)),
        ],
        scratch_shapes=[
            pltpu.VMEM((M, DC_SH), jnp.bfloat16),
            pltpu.VMEM((M, DC_SH), jnp.bfloat16),
            pltpu.VMEM((DC_SH, D), jnp.bfloat16),
            pltpu.VMEM((DC_SH, D), jnp.bfloat16),
            pltpu.SemaphoreType.DMA((3,)),
            pltpu.SemaphoreType.DMA((3,)),
        ],
        compiler_params=_CompilerParams(collective_id=0),
    )(x2bf, wdkv_bf, wuk_bf, wuv_bf, wq)


def _matmul_body(a_ref, w_ref, o_ref, *, out_dtype):
    a = a_ref[...]
    w = w_ref[...]
    if a.dtype != jnp.bfloat16:
        a = a.astype(jnp.bfloat16)
    if w.dtype != jnp.bfloat16:
        w = w.astype(jnp.bfloat16)
    o_ref[...] = jnp.dot(a, w, preferred_element_type=jnp.float32).astype(
        out_dtype
    )


def _matmul(a, w, bn, out_dtype):
    m, k = a.shape
    _, n = w.shape
    assert n % bn == 0
    return pl.pallas_call(
        functools.partial(_matmul_body, out_dtype=out_dtype),
        grid=(n // bn,),
        in_specs=[
            pl.BlockSpec((m, k), lambda j: (0, 0)),
            pl.BlockSpec((k, bn), lambda j: (0, j)),
        ],
        out_specs=pl.BlockSpec((m, bn), lambda j: (0, j)),
        out_shape=jax.ShapeDtypeStruct((m, n), out_dtype),
    )(a, w)


def _attn_body(q_ref, k_ref, qr_ref, kr_ref, v_ref, o_ref):
    kr = kr_ref[0]
    kr_cat = jnp.concatenate([kr, kr], axis=1)
    lane = lax.broadcasted_iota(jnp.int32, (S, 2 * Dr), 1)
    masks = [(lane // Dr) == 0, (lane // Dr) == 1]
    dims = (((1,), (1,)), ((), ()))
    scale = (Dh + Dr) ** -0.5
    for h in range(H):
        q = q_ref[0, :, h * Dh:(h + 1) * Dh]
        k = k_ref[0, :, h * Dh:(h + 1) * Dh]
        v = v_ref[0, :, h * Dh:(h + 1) * Dh]
        qrp = qr_ref[0, :, (h // 2) * 2 * Dr:(h // 2 + 1) * 2 * Dr]
        qr_m = jnp.where(masks[h % 2], qrp, 0).astype(qrp.dtype)
        s = lax.dot_general(q, k, dims, preferred_element_type=jnp.float32)
        s = s + lax.dot_general(
            qr_m, kr_cat, dims, preferred_element_type=jnp.float32
        )
        s = s * scale
        mx = jnp.max(s, axis=1, keepdims=True)
        p = jnp.exp(s - mx)
        p = p * (1.0 / jnp.sum(p, axis=1, keepdims=True))
        o = jnp.dot(
            p.astype(jnp.bfloat16), v, preferred_element_type=jnp.float32
        )
        o_ref[0, :, h * Dh:(h + 1) * Dh] = o.astype(jnp.bfloat16)


def _attention(q3, k3, qr3, kr3, v3):
    return pl.pallas_call(
        _attn_body,
        grid=(B,),
        in_specs=[
            pl.BlockSpec((1, S, H * Dh), lambda b: (b, 0, 0)),
            pl.BlockSpec((1, S, H * Dh), lambda b: (b, 0, 0)),
            pl.BlockSpec((1, S, H * Dr), lambda b: (b, 0, 0)),
            pl.BlockSpec((1, S, Dr), lambda b: (b, 0, 0)),
            pl.BlockSpec((1, S, H * Dh), lambda b: (b, 0, 0)),
        ],
        out_specs=pl.BlockSpec((1, S, H * Dh), lambda b: (b, 0, 0)),
        out_shape=jax.ShapeDtypeStruct((B, S, H * Dh), jnp.bfloat16),
    )(q3, k3, qr3, kr3, v3)


def kernel(x, Wdkv, Wuk, Wuv, Wq, Wqr, Wkr, Wo):
    xbf = x.reshape(M, D).astype(jnp.bfloat16)

    c2, wuk2, wuv2, q2 = _exchange_q(
        xbf,
        Wdkv.astype(jnp.bfloat16),
        Wuk.astype(jnp.bfloat16),
        Wuv.astype(jnp.bfloat16),
        Wq,
    )

    k2 = _matmul(c2, wuk2, 512, jnp.bfloat16)
    v2 = _matmul(c2, wuv2, 512, jnp.bfloat16)
    qr2 = _matmul(xbf, Wqr, 512, jnp.bfloat16)
    kr2 = _matmul(xbf, Wkr, 64, jnp.bfloat16)

    o3 = _attention(
        q2.reshape(B, S, H * Dh),
        k2.reshape(B, S, H * Dh),
        qr2.reshape(B, S, H * Dr),
        kr2.reshape(B, S, Dr),
        v2.reshape(B, S, H * Dh),
    )

    out2 = _matmul(o3.reshape(M, H * Dh), Wo, 512, jnp.float32)
    return out2.reshape(B, S, D)


# device time: 175036 ns/iter; 1.6163x vs baseline; 1.2555x over previous
import functools

import jax
import jax.numpy as jnp
from jax import lax
from jax.experimental import pallas as pl
from jax.experimental.pallas import tpu as pltpu

B, S, H, Dh, Dr = 4, 256, 32, 128, 64
D = 4096
DC = 256
DC_SH = 128
M = B * S

_CompilerParams = getattr(pltpu, "CompilerParams", None) or getattr(
    pltpu, "TPUCompilerParams"
)


_QBLK = 256


def _exq_body(
    x_ref, wdkv_ref, wuk_ref, wuv_ref, wq_ref,
    c_ref, wuk2_ref, wuv2_ref, q_ref,
    c_loc,
    c_rem, wuk_rem, wuv_rem,
    send_sems, recv_sems,
):
    j = pl.program_id(0)
    my_x = lax.axis_index("x")
    my_y = lax.axis_index("y")
    my_z = lax.axis_index("z")
    partner = (1 - my_x, my_y, my_z)

    rdmas = [
        pltpu.make_async_remote_copy(
            src_ref=src,
            dst_ref=dst,
            send_sem=send_sems.at[i],
            recv_sem=recv_sems.at[i],
            device_id=partner,
            device_id_type=pl.DeviceIdType.MESH,
        )
        for i, (src, dst) in enumerate(
            [(wuk_ref, wuk_rem), (wuv_ref, wuv_rem), (c_loc, c_rem)]
        )
    ]

    @pl.when(j == 0)
    def _():
        barrier = pltpu.get_barrier_semaphore()
        pl.semaphore_signal(
            barrier, inc=1, device_id=partner,
            device_id_type=pl.DeviceIdType.MESH,
        )
        pl.semaphore_wait(barrier, 1)

        rdmas[0].start()
        rdmas[1].start()
        c_loc[...] = jnp.dot(
            x_ref[...], wdkv_ref[...], preferred_element_type=jnp.float32
        ).astype(jnp.bfloat16)
        rdmas[2].start()
        c_ref[:, 0:DC_SH] = c_loc[...]
        wuk2_ref[0:DC_SH, :] = wuk_ref[...]
        wuv2_ref[0:DC_SH, :] = wuv_ref[...]

    q_ref[...] = jnp.dot(
        x_ref[...], wq_ref[...].astype(jnp.bfloat16),
        preferred_element_type=jnp.float32,
    ).astype(jnp.bfloat16)

    @pl.when(j == pl.num_programs(0) - 1)
    def _():
        for rdma in rdmas:
            rdma.wait()
        c_ref[:, DC_SH:DC] = c_rem[...]
        wuk2_ref[DC_SH:DC, :] = wuk_rem[...]
        wuv2_ref[DC_SH:DC, :] = wuv_rem[...]


def _exchange_q(x2bf, wdkv_bf, wuk_bf, wuv_bf, wq):
    return pl.pallas_call(
        _exq_body,
        grid=(D // _QBLK,),
        out_shape=[
            jax.ShapeDtypeStruct((M, DC), jnp.bfloat16),
            jax.ShapeDtypeStruct((DC, D), jnp.bfloat16),
            jax.ShapeDtypeStruct((DC, D), jnp.bfloat16),
            jax.ShapeDtypeStruct((M, D), jnp.bfloat16),
        ],
        in_specs=[
            pl.BlockSpec((M, D), lambda j: (0, 0)),
            pl.BlockSpec((D, DC_SH), lambda j: (0, 0)),
            pl.BlockSpec((DC_SH, D), lambda j: (0, 0)),
            pl.BlockSpec((DC_SH, D), lambda j: (0, 0)),
            pl.BlockSpec((D, _QBLK), lambda j: (0, j)),
        ],
        out_specs=[
            pl.BlockSpec((M, DC), lambda j: (0, 0)),
            pl.BlockSpec((DC, D), lambda j: (0, 0)),
            pl.BlockSpec((DC, D), lambda j: (0, 0)),
            pl.BlockSpec((M, _QBLK), lambda j: (0, j)),
        ],
        scratch_shapes=[
            pltpu.VMEM((M, DC_SH), jnp.bfloat16),
            pltpu.VMEM((M, DC_SH), jnp.bfloat16),
            pltpu.VMEM((DC_SH, D), jnp.bfloat16),
            pltpu.VMEM((DC_SH, D), jnp.bfloat16),
            pltpu.SemaphoreType.DMA((3,)),
            pltpu.SemaphoreType.DMA((3,)),
        ],
        compiler_params=_CompilerParams(collective_id=0),
    )(x2bf, wdkv_bf, wuk_bf, wuv_bf, wq)


def _matmul_body(a_ref, w_ref, o_ref, *, out_dtype):
    a = a_ref[...]
    w = w_ref[...]
    if a.dtype != jnp.bfloat16:
        a = a.astype(jnp.bfloat16)
    if w.dtype != jnp.bfloat16:
        w = w.astype(jnp.bfloat16)
    o_ref[...] = jnp.dot(a, w, preferred_element_type=jnp.float32).astype(
        out_dtype
    )


def _matmul(a, w, bn, out_dtype):
    m, k = a.shape
    _, n = w.shape
    assert n % bn == 0
    return pl.pallas_call(
        functools.partial(_matmul_body, out_dtype=out_dtype),
        grid=(n // bn,),
        in_specs=[
            pl.BlockSpec((m, k), lambda j: (0, 0)),
            pl.BlockSpec((k, bn), lambda j: (0, j)),
        ],
        out_specs=pl.BlockSpec((m, bn), lambda j: (0, j)),
        out_shape=jax.ShapeDtypeStruct((m, n), out_dtype),
    )(a, w)


def _attn_body(c_ref, wuk_ref, wuv_ref, q_ref, qr_ref, kr_ref, o_ref,
               k_s, v_s):
    k_s[...] = jnp.dot(
        c_ref[...], wuk_ref[...], preferred_element_type=jnp.float32
    ).astype(jnp.bfloat16)
    v_s[...] = jnp.dot(
        c_ref[...], wuv_ref[...], preferred_element_type=jnp.float32
    ).astype(jnp.bfloat16)

    kr = kr_ref[...]
    kr_cat = jnp.concatenate([kr, kr], axis=1)
    lane = lax.broadcasted_iota(jnp.int32, (S, 2 * Dr), 1)
    scale = (Dh + Dr) ** -0.5
    masks = [(lane // Dr) == 0, (lane // Dr) == 1]
    dims = (((1,), (1,)), ((), ()))
    for h in range(H):
        q = (q_ref[:, h * Dh:(h + 1) * Dh] * scale).astype(jnp.bfloat16)
        k = k_s[:, h * Dh:(h + 1) * Dh]
        v = v_s[:, h * Dh:(h + 1) * Dh]
        qrp = qr_ref[:, (h // 2) * 2 * Dr:(h // 2 + 1) * 2 * Dr]
        qr_m = jnp.where(masks[h % 2], qrp * scale, 0).astype(jnp.bfloat16)
        s = lax.dot_general(q, k, dims, preferred_element_type=jnp.float32)
        s = s + lax.dot_general(
            qr_m, kr_cat, dims, preferred_element_type=jnp.float32
        )
        p = jnp.exp(s)
        denom = jnp.sum(p, axis=1, keepdims=True)
        o = jnp.dot(
            p.astype(jnp.bfloat16), v, preferred_element_type=jnp.float32
        )
        o_ref[:, h * Dh:(h + 1) * Dh] = (o * (1.0 / denom)).astype(
            jnp.bfloat16
        )


def _attention(c2, wuk2, wuv2, q2, qr2, kr2):
    return pl.pallas_call(
        _attn_body,
        grid=(B,),
        in_specs=[
            pl.BlockSpec((S, DC), lambda b: (b, 0)),
            pl.BlockSpec((DC, D), lambda b: (0, 0)),
            pl.BlockSpec((DC, D), lambda b: (0, 0)),
            pl.BlockSpec((S, D), lambda b: (b, 0)),
            pl.BlockSpec((S, H * Dr), lambda b: (b, 0)),
            pl.BlockSpec((S, Dr), lambda b: (b, 0)),
        ],
        out_specs=pl.BlockSpec((S, H * Dh), lambda b: (b, 0)),
        out_shape=jax.ShapeDtypeStruct((M, H * Dh), jnp.bfloat16),
        scratch_shapes=[
            pltpu.VMEM((S, D), jnp.bfloat16),
            pltpu.VMEM((S, D), jnp.bfloat16),
        ],
    )(c2, wuk2, wuv2, q2, qr2, kr2)


def kernel(x, Wdkv, Wuk, Wuv, Wq, Wqr, Wkr, Wo):
    xbf = x.reshape(M, D).astype(jnp.bfloat16)

    c2, wuk2, wuv2, q2 = _exchange_q(
        xbf,
        Wdkv.astype(jnp.bfloat16),
        Wuk.astype(jnp.bfloat16),
        Wuv.astype(jnp.bfloat16),
        Wq,
    )

    qr2 = _matmul(xbf, Wqr, 512, jnp.bfloat16)
    kr2 = _matmul(xbf, Wkr, 64, jnp.bfloat16)

    o2 = _attention(c2, wuk2, wuv2, q2, qr2, kr2)

    out2 = _matmul(o2, Wo, 512, jnp.float32)
    return out2.reshape(B, S, D)
